# baseline (speedup 1.0000x reference)
import functools

import jax
import jax.numpy as jnp
from jax import lax
from jax.experimental import pallas as pl
from jax.experimental.pallas import tpu as pltpu
from jax.experimental.pallas import tpu_sc as plsc

VOCAB = 1000
B_SZ = 1024
T_SZ = 50
NTOK = B_SZ * T_SZ
NW = 32
TPW = NTOK // NW
RC = T_SZ
NRC = TPW // RC
SC_CH = 64
N_SCCH = TPW // SC_CH

_MESH = plsc.VectorSubcoreMesh(core_axis_name="c", subcore_axis_name="s")
_SC_PARAMS = pltpu.CompilerParams(use_tc_tiling_on_sc=False)


def _row_lse_body(table_ref, out_ref):
    t = table_ref[...]
    m = jnp.max(t, axis=1, keepdims=True)
    s = jnp.sum(jnp.exp(t - m), axis=1, keepdims=True)
    out_ref[...] = jnp.log(s) + m


def _row_lse(table):
    out = pl.pallas_call(
        _row_lse_body,
        out_shape=jax.ShapeDtypeStruct((VOCAB, 1), jnp.float32),
    )(table)
    return out.reshape(VOCAB)


@functools.partial(
    pl.kernel,
    mesh=_MESH,
    compiler_params=_SC_PARAMS,
    out_type=[
        jax.ShapeDtypeStruct((B_SZ, T_SZ, VOCAB), jnp.float32),
        jax.ShapeDtypeStruct((NW, 16), jnp.float32),
    ],
    scratch_types=[
        pltpu.VMEM((NRC, RC), jnp.int32),
        pltpu.VMEM((TPW,), jnp.int32),
        pltpu.VMEM((TPW,), jnp.int32),
        pltpu.VMEM((TPW,), jnp.int32),
        pltpu.VMEM((TPW,), jnp.float32),
        pltpu.VMEM((TPW,), jnp.float32),
        pltpu.VMEM((RC, VOCAB), jnp.float32),
        pltpu.VMEM((RC, VOCAB), jnp.float32),
        pltpu.VMEM((16,), jnp.float32),
        pltpu.SemaphoreType.DMA,
        pltpu.SemaphoreType.DMA,
        pltpu.SemaphoreType.DMA,
        pltpu.SemaphoreType.DMA,
    ],
)
def _sc_main(x2_hbm, xf_hbm, y_hbm, table_hbm, tflat_hbm, lse_hbm,
             out_hbm, part_hbm,
             xr, xs, yv, fv, lsev, pick, buf0, buf1, acc_v,
             sg0, sg1, sl, sp):
    cid = lax.axis_index("c")
    sid = lax.axis_index("s")
    wid = sid * 2 + cid
    base = wid * TPW
    batch0 = wid * NRC

    pltpu.sync_copy(x2_hbm.at[wid], xr)
    pltpu.sync_copy(xf_hbm.at[pl.ds(base, TPW)], xs)
    pltpu.sync_copy(y_hbm.at[pl.ds(base, TPW)], yv)

    def build_flat(i, carry):
        s16 = pl.ds(i * 16, 16)
        fv[s16] = xs[s16] * VOCAB + yv[s16]
        return carry

    lax.fori_loop(0, TPW // 16, build_flat, 0)

    def scalar_desc(i):
        s = pl.ds(i * SC_CH, SC_CH)
        dl = pltpu.make_async_copy(lse_hbm.at[xs.at[s]], lsev.at[s], sl)
        dp = pltpu.make_async_copy(tflat_hbm.at[fv.at[s]], pick.at[s], sp)
        return dl, dp

    def fire(i, carry):
        dl, dp = scalar_desc(i)
        dl.start()
        dp.start()
        return carry

    lax.fori_loop(0, N_SCCH, fire, 0)

    def gather_start(j, buf, sem):
        return pltpu.make_async_copy(table_hbm.at[xr.at[j]], buf, sem)

    gather_start(0, buf0, sg0).start()

    def body(g, carry):
        j = 2 * g
        gather_start(j, buf0, sg0).wait()
        gather_start(j + 1, buf1, sg1).start()
        pltpu.sync_copy(buf0, out_hbm.at[batch0 + j])
        gather_start(j + 1, buf1, sg1).wait()

        @pl.when(j + 2 < NRC)
        def _():
            gather_start(j + 2, buf0, sg0).start()

        pltpu.sync_copy(buf1, out_hbm.at[batch0 + j + 1])
        return carry

    lax.fori_loop(0, NRC // 2, body, 0)

    def drain(i, carry):
        dl, dp = scalar_desc(i)
        dl.wait()
        dp.wait()
        return carry

    lax.fori_loop(0, N_SCCH, drain, 0)

    def accum(i, a):
        s16 = pl.ds(i * 16, 16)
        return a + (lsev[s16] - pick[s16])

    acc = lax.fori_loop(0, TPW // 16, accum,
                        jnp.zeros((16,), jnp.float32))
    acc_v[...] = acc
    pltpu.sync_copy(acc_v, part_hbm.at[wid])


def _loss_body(part_ref, out_ref):
    out_ref[...] = jnp.sum(part_ref[...], keepdims=True) / NTOK


def _final_loss(partials):
    out = pl.pallas_call(
        _loss_body,
        out_shape=jax.ShapeDtypeStruct((1, 1), jnp.float32),
    )(partials)
    return out[0, 0]


def kernel(x, y, table):
    x32 = x.astype(jnp.int32)
    y32 = y.reshape(-1).astype(jnp.int32)
    table = table.astype(jnp.float32)
    tflat = jnp.concatenate(
        [table.reshape(-1), jnp.zeros((8,), jnp.float32)])
    row_lse = _row_lse(table)
    logits, partials = _sc_main(
        x32.reshape(NW, NRC, RC), x32.reshape(-1), y32,
        table, tflat, row_lse)
    loss = _final_loss(partials)
    return (logits, loss)
